# baseline (device time: 40329 ns/iter reference)
import jax
import jax.numpy as jnp
from jax import lax
from jax.experimental import pallas as pl
from jax.experimental.pallas import tpu as pltpu

B, SQ, SKV, H, D = 8, 1, 512, 8, 64


def kernel(Q, K, V):
    scale = D ** -0.5

    def body(q_ref, k_ref, v_ref, out_ref, send_buf, recv_buf, send_sem, recv_sem):
        my_x = lax.axis_index("x")
        my_y = lax.axis_index("y")
        my_z = lax.axis_index("z")
        nbr = (my_x, 1 - my_y, my_z)

        barrier_sem = pltpu.get_barrier_semaphore()
        pl.semaphore_signal(
            barrier_sem, inc=1, device_id=nbr, device_id_type=pl.DeviceIdType.MESH
        )
        pl.semaphore_wait(barrier_sem, 1)

        q = q_ref[:, 0, :, :]
        k = k_ref[...]
        v = v_ref[...]
        s = jnp.sum(q[:, None, :, :] * k, axis=-1) * scale
        m = jnp.max(s, axis=1)
        p = jnp.exp(s - m[:, None, :])
        l = jnp.sum(p, axis=1)
        o = jnp.sum(p[:, :, :, None] * v, axis=1)

        send_buf[:, :, 0:D] = o
        send_buf[:, :, D:2 * D] = jnp.broadcast_to(m[:, :, None], (B, H, D))
        send_buf[:, :, 2 * D:3 * D] = jnp.broadcast_to(l[:, :, None], (B, H, D))

        rdma = pltpu.make_async_remote_copy(
            src_ref=send_buf,
            dst_ref=recv_buf,
            send_sem=send_sem,
            recv_sem=recv_sem,
            device_id=nbr,
            device_id_type=pl.DeviceIdType.MESH,
        )
        rdma.start()
        rdma.wait()

        o2 = recv_buf[:, :, 0:D]
        m2 = recv_buf[:, :, D:2 * D]
        l2 = recv_buf[:, :, 2 * D:3 * D]
        m1 = jnp.broadcast_to(m[:, :, None], (B, H, D))
        l1 = jnp.broadcast_to(l[:, :, None], (B, H, D))
        mn = jnp.maximum(m1, m2)
        a1 = jnp.exp(m1 - mn)
        a2 = jnp.exp(m2 - mn)
        out_ref[:, 0, :, :] = (a1 * o + a2 * o2) / (a1 * l1 + a2 * l2)

    return pl.pallas_call(
        body,
        out_shape=jax.ShapeDtypeStruct((B, SQ, H, D), jnp.float32),
        in_specs=[
            pl.BlockSpec(memory_space=pltpu.VMEM),
            pl.BlockSpec(memory_space=pltpu.VMEM),
            pl.BlockSpec(memory_space=pltpu.VMEM),
        ],
        out_specs=pl.BlockSpec(memory_space=pltpu.VMEM),
        scratch_shapes=[
            pltpu.VMEM((B, H, 3 * D), jnp.float32),
            pltpu.VMEM((B, H, 3 * D), jnp.float32),
            pltpu.SemaphoreType.DMA,
            pltpu.SemaphoreType.DMA,
        ],
        compiler_params=pltpu.CompilerParams(collective_id=0),
    )(Q, K, V)


# device time: 38451 ns/iter; 1.0488x vs baseline; 1.0488x over previous
import jax
import jax.numpy as jnp
from jax import lax
from jax.experimental import pallas as pl
from jax.experimental.pallas import tpu as pltpu

B, SQ, SKV, H, D = 8, 1, 512, 8, 64


def kernel(Q, K, V):
    scale = D ** -0.5

    def body(q_ref, k_ref, v_ref, out_ref, send_buf, recv_buf, send_sem, recv_sem):
        my_x = lax.axis_index("x")
        my_y = lax.axis_index("y")
        my_z = lax.axis_index("z")
        nbr = (my_x, 1 - my_y, my_z)

        barrier_sem = pltpu.get_barrier_semaphore()
        pl.semaphore_signal(
            barrier_sem, inc=1, device_id=nbr, device_id_type=pl.DeviceIdType.MESH
        )
        pl.semaphore_wait(barrier_sem, 1)

        q = q_ref[:, 0, :, :]
        k = k_ref[...]
        v = v_ref[...]
        s = jnp.sum(q[:, None, :, :] * k, axis=-1) * scale
        m = jnp.max(s, axis=1)
        p = jnp.exp(s - m[:, None, :])
        l = jnp.sum(p, axis=1)
        o = jnp.sum(p[:, :, :, None] * v, axis=1)

        out_ref[:, 0, :, :] = o / l[:, :, None]
        return

        send_buf[:, :, 0:D] = o
        send_buf[:, :, D:2 * D] = jnp.broadcast_to(m[:, :, None], (B, H, D))
        send_buf[:, :, 2 * D:3 * D] = jnp.broadcast_to(l[:, :, None], (B, H, D))

        rdma = pltpu.make_async_remote_copy(
            src_ref=send_buf,
            dst_ref=recv_buf,
            send_sem=send_sem,
            recv_sem=recv_sem,
            device_id=nbr,
            device_id_type=pl.DeviceIdType.MESH,
        )
        rdma.start()
        rdma.wait()

        o2 = recv_buf[:, :, 0:D]
        m2 = recv_buf[:, :, D:2 * D]
        l2 = recv_buf[:, :, 2 * D:3 * D]
        m1 = jnp.broadcast_to(m[:, :, None], (B, H, D))
        l1 = jnp.broadcast_to(l[:, :, None], (B, H, D))
        mn = jnp.maximum(m1, m2)
        a1 = jnp.exp(m1 - mn)
        a2 = jnp.exp(m2 - mn)
        out_ref[:, 0, :, :] = (a1 * o + a2 * o2) / (a1 * l1 + a2 * l2)

    return pl.pallas_call(
        body,
        out_shape=jax.ShapeDtypeStruct((B, SQ, H, D), jnp.float32),
        in_specs=[
            pl.BlockSpec(memory_space=pltpu.VMEM),
            pl.BlockSpec(memory_space=pltpu.VMEM),
            pl.BlockSpec(memory_space=pltpu.VMEM),
        ],
        out_specs=pl.BlockSpec(memory_space=pltpu.VMEM),
        scratch_shapes=[
            pltpu.VMEM((B, H, 3 * D), jnp.float32),
            pltpu.VMEM((B, H, 3 * D), jnp.float32),
            pltpu.SemaphoreType.DMA,
            pltpu.SemaphoreType.DMA,
        ],
        compiler_params=pltpu.CompilerParams(collective_id=0),
    )(Q, K, V)


# device time: 28481 ns/iter; 1.4160x vs baseline; 1.3501x over previous
import jax
import jax.numpy as jnp
from jax import lax
from jax.experimental import pallas as pl
from jax.experimental.pallas import tpu as pltpu

B, SQ, SKV, H, D = 8, 1, 512, 8, 64


def kernel(Q, K, V):
    scale = D ** -0.5

    def body(q_ref, k_ref, v_ref, out_ref, send_buf, recv_buf, send_sem, recv_sem):
        my_x = lax.axis_index("x")
        my_y = lax.axis_index("y")
        my_z = lax.axis_index("z")
        nbr = (my_x, 1 - my_y, my_z)

        barrier_sem = pltpu.get_barrier_semaphore()
        pl.semaphore_signal(
            barrier_sem, inc=1, device_id=nbr, device_id_type=pl.DeviceIdType.MESH
        )
        pl.semaphore_wait(barrier_sem, 1)

        out_ref[:, 0, :, :] = q_ref[:, 0, :, :] + k_ref[0, 0, 0, 0] + v_ref[0, 0, 0, 0]
        return

        q = q_ref[:, 0, :, :]
        k = k_ref[...]
        v = v_ref[...]
        s = jnp.sum(q[:, None, :, :] * k, axis=-1) * scale
        m = jnp.max(s, axis=1)
        p = jnp.exp(s - m[:, None, :])
        l = jnp.sum(p, axis=1)
        o = jnp.sum(p[:, :, :, None] * v, axis=1)

        out_ref[:, 0, :, :] = o / l[:, :, None]
        return

        send_buf[:, :, 0:D] = o
        send_buf[:, :, D:2 * D] = jnp.broadcast_to(m[:, :, None], (B, H, D))
        send_buf[:, :, 2 * D:3 * D] = jnp.broadcast_to(l[:, :, None], (B, H, D))

        rdma = pltpu.make_async_remote_copy(
            src_ref=send_buf,
            dst_ref=recv_buf,
            send_sem=send_sem,
            recv_sem=recv_sem,
            device_id=nbr,
            device_id_type=pl.DeviceIdType.MESH,
        )
        rdma.start()
        rdma.wait()

        o2 = recv_buf[:, :, 0:D]
        m2 = recv_buf[:, :, D:2 * D]
        l2 = recv_buf[:, :, 2 * D:3 * D]
        m1 = jnp.broadcast_to(m[:, :, None], (B, H, D))
        l1 = jnp.broadcast_to(l[:, :, None], (B, H, D))
        mn = jnp.maximum(m1, m2)
        a1 = jnp.exp(m1 - mn)
        a2 = jnp.exp(m2 - mn)
        out_ref[:, 0, :, :] = (a1 * o + a2 * o2) / (a1 * l1 + a2 * l2)

    return pl.pallas_call(
        body,
        out_shape=jax.ShapeDtypeStruct((B, SQ, H, D), jnp.float32),
        in_specs=[
            pl.BlockSpec(memory_space=pltpu.VMEM),
            pl.BlockSpec(memory_space=pltpu.VMEM),
            pl.BlockSpec(memory_space=pltpu.VMEM),
        ],
        out_specs=pl.BlockSpec(memory_space=pltpu.VMEM),
        scratch_shapes=[
            pltpu.VMEM((B, H, 3 * D), jnp.float32),
            pltpu.VMEM((B, H, 3 * D), jnp.float32),
            pltpu.SemaphoreType.DMA,
            pltpu.SemaphoreType.DMA,
        ],
        compiler_params=pltpu.CompilerParams(collective_id=0),
    )(Q, K, V)


# device time: 27018 ns/iter; 1.4927x vs baseline; 1.0541x over previous
import jax
import jax.numpy as jnp
from jax import lax
from jax.experimental import pallas as pl
from jax.experimental.pallas import tpu as pltpu

B, SQ, SKV, H, D = 8, 1, 512, 8, 64
NSLOT = 4


def kernel(Q, K, V):
    def body(q_ref, k_ref, v_ref, out_ref, kbuf, vbuf, sems):
        acc = q_ref[:, 0, :, :]
        for rnd in range(B // NSLOT):
            copies = []
            for s in range(NSLOT):
                b = rnd * NSLOT + s
                kc = pltpu.make_async_copy(k_ref.at[b], kbuf.at[s], sems.at[s])
                vc = pltpu.make_async_copy(v_ref.at[b], vbuf.at[s], sems.at[NSLOT + s])
                kc.start()
                vc.start()
                copies.append((kc, vc))
            for kc, vc in copies:
                kc.wait()
                vc.wait()
            acc = acc + kbuf[0, 0, 0, 0] + vbuf[0, 0, 0, 0]
        out_ref[:, 0, :, :] = acc

    return pl.pallas_call(
        body,
        out_shape=jax.ShapeDtypeStruct((B, SQ, H, D), jnp.float32),
        in_specs=[
            pl.BlockSpec(memory_space=pltpu.MemorySpace.VMEM),
            pl.BlockSpec(memory_space=pl.ANY),
            pl.BlockSpec(memory_space=pl.ANY),
        ],
        out_specs=pl.BlockSpec(memory_space=pltpu.MemorySpace.VMEM),
        scratch_shapes=[
            pltpu.VMEM((NSLOT, SKV, H, D), jnp.float32),
            pltpu.VMEM((NSLOT, SKV, H, D), jnp.float32),
            pltpu.SemaphoreType.DMA((2 * NSLOT,)),
        ],
    )(Q, K, V)


# device time: 14098 ns/iter; 2.8606x vs baseline; 1.9164x over previous
import jax
import jax.numpy as jnp
from jax import lax
from jax.experimental import pallas as pl
from jax.experimental.pallas import tpu as pltpu

B, SQ, SKV, H, D = 8, 1, 512, 8, 64


def kernel(Q, K, V):
    scale = D ** -0.5
    Kt = jnp.transpose(K, (0, 2, 3, 1))
    Vt = jnp.transpose(V, (0, 2, 3, 1))

    def body(q_ref, k_ref, v_ref, out_ref, kbuf, vbuf,
             send_buf, recv_buf, sems, send_sem, recv_sem):
        my_x = lax.axis_index("x")
        my_y = lax.axis_index("y")
        my_z = lax.axis_index("z")
        nbr = (my_x, 1 - my_y, my_z)

        barrier_sem = pltpu.get_barrier_semaphore()
        pl.semaphore_signal(
            barrier_sem, inc=1, device_id=nbr, device_id_type=pl.DeviceIdType.MESH
        )

        copies = []
        for b in range(B):
            kc = pltpu.make_async_copy(k_ref.at[b], kbuf.at[b], sems.at[b])
            vc = pltpu.make_async_copy(v_ref.at[b], vbuf.at[b], sems.at[B + b])
            kc.start()
            vc.start()
            copies.append((kc, vc))

        pl.semaphore_wait(barrier_sem, 1)

        for b in range(B):
            kc, vc = copies[b]
            q_b = q_ref[b, 0]
            kc.wait()
            s = jnp.sum(q_b[:, :, None] * kbuf[b], axis=1) * scale
            m = jnp.max(s, axis=-1, keepdims=True)
            p = jnp.exp(s - m)
            l = jnp.sum(p, axis=-1, keepdims=True)
            vc.wait()
            o = jnp.sum(p[:, None, :] * vbuf[b], axis=-1)
            send_buf[b, :, 0:D] = o
            send_buf[b, :, D:2 * D] = jnp.broadcast_to(m, (H, D))
            send_buf[b, :, 2 * D:3 * D] = jnp.broadcast_to(l, (H, D))

        rdma = pltpu.make_async_remote_copy(
            src_ref=send_buf,
            dst_ref=recv_buf,
            send_sem=send_sem,
            recv_sem=recv_sem,
            device_id=nbr,
            device_id_type=pl.DeviceIdType.MESH,
        )
        rdma.start()
        rdma.wait()

        o1 = send_buf[:, :, 0:D]
        m1 = send_buf[:, :, D:2 * D]
        l1 = send_buf[:, :, 2 * D:3 * D]
        o2 = recv_buf[:, :, 0:D]
        m2 = recv_buf[:, :, D:2 * D]
        l2 = recv_buf[:, :, 2 * D:3 * D]
        mn = jnp.maximum(m1, m2)
        a1 = jnp.exp(m1 - mn)
        a2 = jnp.exp(m2 - mn)
        out_ref[:, 0, :, :] = (a1 * o1 + a2 * o2) / (a1 * l1 + a2 * l2)

    return pl.pallas_call(
        body,
        out_shape=jax.ShapeDtypeStruct((B, SQ, H, D), jnp.float32),
        in_specs=[
            pl.BlockSpec(memory_space=pltpu.MemorySpace.VMEM),
            pl.BlockSpec(memory_space=pl.ANY),
            pl.BlockSpec(memory_space=pl.ANY),
        ],
        out_specs=pl.BlockSpec(memory_space=pltpu.MemorySpace.VMEM),
        scratch_shapes=[
            pltpu.VMEM((B, H, D, SKV), jnp.float32),
            pltpu.VMEM((B, H, D, SKV), jnp.float32),
            pltpu.VMEM((B, H, 3 * D), jnp.float32),
            pltpu.VMEM((B, H, 3 * D), jnp.float32),
            pltpu.SemaphoreType.DMA((2 * B,)),
            pltpu.SemaphoreType.DMA,
            pltpu.SemaphoreType.DMA,
        ],
        compiler_params=pltpu.CompilerParams(
            collective_id=0,
            vmem_limit_bytes=96 * 1024 * 1024,
        ),
    )(Q, Kt, Vt)
